# baseline (speedup 1.0000x reference)
import jax
import jax.numpy as jnp
from jax import lax
from jax.experimental import pallas as pl
from jax.experimental.pallas import tpu as pltpu
from jax.experimental.pallas import tpu_sc as plsc

M = 524288
D = 64
B = 65536

NC = 2
NS = 16
NPASS = 32
RANGE = M // (NPASS * NC)
RSHIFT = 13
TROWS = RANGE // NS
SLICE = B // NS
CHUNKS = SLICE // 16
GARBAGE = RANGE
DUMPBIN = NPASS
LISTCAP = SLICE + 144
W = 128
WV = W // 16


def _body(mem_hbm, val_hbm, idx_hbm, out_hbm, idx_buf, lid_s, pos_s, fill,
          lid_win, pos_win, rows_buf, acc_a, acc_b, ss_a, ss_b, ws_a, ws_b,
          gsem):
    c = lax.axis_index("c")
    s = lax.axis_index("s")

    accs = (acc_a, acc_b)
    ssems = (ss_a, ss_b)
    wsems = (ws_a, ws_b)

    pltpu.sync_copy(idx_hbm.at[pl.ds(s * SLICE, SLICE)], idx_buf)

    lanes = lax.iota(jnp.int32, 16)
    ones = jnp.ones((16,), jnp.int32)
    zeros = jnp.zeros((16,), jnp.int32)

    def stage(p):
        base = (p * NC + c) * RANGE
        a = accs[p % 2]
        return pltpu.async_copy(
            mem_hbm.at[pl.ds(base + s * TROWS, TROWS)],
            a.at[pl.ds(s * TROWS, TROWS)], ssems[p % 2])

    stage_desc = {0: stage(0)}

    def bins_of(i):
        idxv = idx_buf[pl.ds(i * 16, 16)]
        r = idxv >> RSHIFT
        cm = (r & 1) ^ c
        pb = r >> 1
        return idxv, pb * (1 - cm) + DUMPBIN * cm

    fill[pl.ds(0, 16)] = zeros
    fill[pl.ds(16, 16)] = zeros
    fill[pl.ds(32, 16)] = zeros

    def count_chunk(i, _):
        _, binv = bins_of(i)
        plsc.addupdate_scatter(fill, [binv], ones)
        return 0

    with jax.named_scope("bincount"):
        lax.fori_loop(0, CHUNKS, count_chunk, 0)

    f0 = fill[pl.ds(0, 16)]
    e0 = plsc.cumsum(f0) - f0
    t0 = jnp.sum(f0)
    f1 = fill[pl.ds(16, 16)]
    e1 = t0 + plsc.cumsum(f1) - f1
    t1 = t0 + jnp.sum(f1)
    f2 = fill[pl.ds(32, 16)]
    e2 = t1 + plsc.cumsum(f2) - f2
    fill[pl.ds(0, 16)] = e0
    fill[pl.ds(16, 16)] = e1
    fill[pl.ds(32, 16)] = e2

    def scat_chunk(i, _):
        idxv, binv = bins_of(i)
        occ, _ = plsc.scan_count(binv)
        bf = plsc.load_gather(fill, [binv])
        off = bf + occ - 1
        plsc.store_scatter(lid_s, [off], idxv & (RANGE - 1))
        plsc.store_scatter(pos_s, [off], s * SLICE + i * 16 + lanes)
        plsc.addupdate_scatter(fill, [binv], ones)
        return 0

    with jax.named_scope("binscat"):
        lax.fori_loop(0, CHUNKS, scat_chunk, 0)
    fe0 = fill[pl.ds(0, 16)]
    fe1 = fill[pl.ds(16, 16)]

    def endof(b):
        return fe0[b] if b < 16 else fe1[b - 16]

    def adds(p, cur):
        end = endof(p)
        start = endof(p - 1) if p > 0 else jnp.int32(0)
        head = start & 7
        wstart = start - head
        total = head + (end - start)
        nch = (total + (W - 1)) // W

        def win(k, _):
            woff = pl.multiple_of(wstart + k * W, 8)
            for w in range(WV):
                g = k * W + w * 16 + lanes
                valid = (1 - (((g - head) >> 31) & 1)) * \
                        (((g - total) >> 31) & 1)
                lw = lid_s[pl.ds(woff + w * 16, 16)]
                pw = pos_s[pl.ds(woff + w * 16, 16)]
                lid_win[pl.ds(w * 16, 16)] = valid * lw + \
                    (1 - valid) * GARBAGE
                pos_win[pl.ds(w * 16, 16)] = valid * pw
            pltpu.async_copy(val_hbm.at[pos_win], rows_buf, gsem).wait()
            pltpu.sync_copy(rows_buf, cur.at[lid_win], add=True)
            return 0

        lax.fori_loop(0, nch, win, 0)

    wb_desc = {}
    for p in range(NPASS):
        cur = accs[p % 2]
        base = (p * NC + c) * RANGE
        with jax.named_scope("wbwait"):
            if p + 1 < NPASS:
                if p >= 1:
                    wb_desc[p - 1].wait()
                stage_desc[p + 1] = stage(p + 1)
        with jax.named_scope("stwait"):
            stage_desc[p].wait()
        with jax.named_scope("bar1"):
            plsc.subcore_barrier()
        with jax.named_scope("adds"):
            adds(p, cur)
        with jax.named_scope("bar2"):
            plsc.subcore_barrier()
        wb_desc[p] = pltpu.async_copy(
            cur.at[pl.ds(s * TROWS, TROWS)],
            out_hbm.at[pl.ds(base + s * TROWS, TROWS)], wsems[p % 2])
    wb_desc[NPASS - 1].wait()


@jax.jit
def _scatter_add(mem, val, idx):
    mesh = plsc.VectorSubcoreMesh(core_axis_name="c", subcore_axis_name="s")
    return pl.kernel(
        _body,
        out_type=jax.ShapeDtypeStruct((M, D), jnp.float32),
        mesh=mesh,
        compiler_params=pltpu.CompilerParams(needs_layout_passes=False,
                                             use_tc_tiling_on_sc=False),
        scratch_types=[
            pltpu.VMEM((SLICE,), jnp.int32),
            pltpu.VMEM((LISTCAP,), jnp.int32),
            pltpu.VMEM((LISTCAP,), jnp.int32),
            pltpu.VMEM((48,), jnp.int32),
            pltpu.VMEM((W,), jnp.int32),
            pltpu.VMEM((W,), jnp.int32),
            pltpu.VMEM((W, D), jnp.float32),
            pltpu.VMEM_SHARED((RANGE + 8, D), jnp.float32),
            pltpu.VMEM_SHARED((RANGE + 8, D), jnp.float32),
            pltpu.SemaphoreType.DMA,
            pltpu.SemaphoreType.DMA,
            pltpu.SemaphoreType.DMA,
            pltpu.SemaphoreType.DMA,
            pltpu.SemaphoreType.DMA,
        ],
    )(mem, val, idx)


def kernel(mem, val, idx):
    return _scatter_add(mem, val, idx)


# speedup vs baseline: 1.0038x; 1.0038x over previous
import jax
import jax.numpy as jnp
from jax import lax
from jax.experimental import pallas as pl
from jax.experimental.pallas import tpu as pltpu
from jax.experimental.pallas import tpu_sc as plsc

M = 524288
D = 64
B = 65536

NC = 2
NS = 16
NPASS = 32
RANGE = M // (NPASS * NC)
RSHIFT = 13
TROWS = RANGE // NS
SLICE = B // NS
CHUNKS = SLICE // 16
GARBAGE = RANGE
DUMPBIN = NPASS
LISTCAP = SLICE + 144
W = 128
NBUF = 4
WV = W // 16


def _body(mem_hbm, val_hbm, idx_hbm, out_hbm, idx_buf, lid_s, pos_s, fill,
          lw0, lw1, lw2, lw3, pw0, pw1, pw2, pw3, rb0, rb1, rb2, rb3,
          acc_a, acc_b, ss_a, ss_b, ws_a, ws_b, gs0, gs1, gs2, gs3):
    c = lax.axis_index("c")
    s = lax.axis_index("s")

    accs = (acc_a, acc_b)
    ssems = (ss_a, ss_b)
    wsems = (ws_a, ws_b)
    lid_wins = (lw0, lw1, lw2, lw3)
    pos_wins = (pw0, pw1, pw2, pw3)
    rows_bufs = (rb0, rb1, rb2, rb3)
    gsems = (gs0, gs1, gs2, gs3)

    pltpu.sync_copy(idx_hbm.at[pl.ds(s * SLICE, SLICE)], idx_buf)

    lanes = lax.iota(jnp.int32, 16)
    ones = jnp.ones((16,), jnp.int32)
    zeros = jnp.zeros((16,), jnp.int32)

    def stage_copy(p, half):
        base = (p * NC + c) * RANGE
        return pltpu.make_async_copy(
            mem_hbm.at[pl.ds(base + s * TROWS, TROWS)],
            accs[half].at[pl.ds(s * TROWS, TROWS)], ssems[half])

    def wb_copy(p, half):
        base = (p * NC + c) * RANGE
        return pltpu.make_async_copy(
            accs[half].at[pl.ds(s * TROWS, TROWS)],
            out_hbm.at[pl.ds(base + s * TROWS, TROWS)], wsems[half])

    stage_copy(0, 0).start()

    def bins_of(i):
        idxv = idx_buf[pl.ds(i * 16, 16)]
        r = idxv >> RSHIFT
        cm = (r & 1) ^ c
        pb = r >> 1
        return idxv, pb * (1 - cm) + DUMPBIN * cm

    fill[pl.ds(0, 16)] = zeros
    fill[pl.ds(16, 16)] = zeros
    fill[pl.ds(32, 16)] = zeros

    def count_chunk(i, _):
        _, binv = bins_of(i)
        plsc.addupdate_scatter(fill, [binv], ones)
        return 0

    with jax.named_scope("bincount"):
        lax.fori_loop(0, CHUNKS, count_chunk, 0)

    f0 = fill[pl.ds(0, 16)]
    e0 = plsc.cumsum(f0) - f0
    t0 = jnp.sum(f0)
    f1 = fill[pl.ds(16, 16)]
    e1 = t0 + plsc.cumsum(f1) - f1
    t1 = t0 + jnp.sum(f1)
    f2 = fill[pl.ds(32, 16)]
    e2 = t1 + plsc.cumsum(f2) - f2
    fill[pl.ds(0, 16)] = e0
    fill[pl.ds(16, 16)] = e1
    fill[pl.ds(32, 16)] = e2

    def scat_chunk(i, _):
        idxv, binv = bins_of(i)
        occ, _ = plsc.scan_count(binv)
        bf = plsc.load_gather(fill, [binv])
        off = bf + occ - 1
        plsc.store_scatter(lid_s, [off], idxv & (RANGE - 1))
        plsc.store_scatter(pos_s, [off], s * SLICE + i * 16 + lanes)
        plsc.addupdate_scatter(fill, [binv], ones)
        return 0

    with jax.named_scope("binscat"):
        lax.fori_loop(0, CHUNKS, scat_chunk, 0)

    def endof(b):
        bv = zeros + b
        return plsc.load_gather(fill, [bv])[0]

    def adds(p, cur):
        end = endof(p)
        start = endof(jnp.maximum(p - 1, 0)) * jnp.minimum(p, 1)
        head = start & 7
        wstart = start - head
        total = head + (end - start)
        nch = (total + (W - 1)) // W
        ngr = (nch + (NBUF - 1)) // NBUF

        def prep(k, b):
            def mrow(w, _):
                woff = pl.multiple_of(wstart + k * W + w * 16, 8)
                g = k * W + w * 16 + lanes
                valid = (1 - (((g - head) >> 31) & 1)) * \
                        (((g - total) >> 31) & 1)
                lw = lid_s[pl.ds(woff, 16)]
                pw = pos_s[pl.ds(woff, 16)]
                lid_wins[b][pl.ds(w * 16, 16)] = valid * lw + \
                    (1 - valid) * GARBAGE
                pos_wins[b][pl.ds(w * 16, 16)] = valid * pw
                return 0

            lax.fori_loop(0, WV, mrow, 0)

        def grp(gidx, _):
            for b in range(NBUF):
                k = gidx * NBUF + b

                @pl.when(k < nch)
                def _fire(k=k, b=b):
                    prep(k, b)
                    pltpu.async_copy(val_hbm.at[pos_wins[b]], rows_bufs[b],
                                     gsems[b])

            for b in range(NBUF):
                k = gidx * NBUF + b

                @pl.when(k < nch)
                def _drain(b=b):
                    pltpu.make_async_copy(val_hbm.at[pos_wins[b]],
                                          rows_bufs[b], gsems[b]).wait()
                    pltpu.sync_copy(rows_bufs[b], cur.at[lid_wins[b]],
                                    add=True)
            return 0

        lax.fori_loop(0, ngr, grp, 0)

    def pass_body(p, half):
        cur = accs[half]
        other = 1 - half
        with jax.named_scope("wbwait"):
            @pl.when(p >= 1)
            def _():
                wb_copy(p - 1, other).wait()

            @pl.when(p + 1 < NPASS)
            def _():
                stage_copy(p + 1, other).start()
        with jax.named_scope("stwait"):
            stage_copy(p, half).wait()
        with jax.named_scope("bar1"):
            plsc.subcore_barrier()
        with jax.named_scope("adds"):
            adds(p, cur)
        with jax.named_scope("bar2"):
            plsc.subcore_barrier()
        wb_copy(p, half).start()

    def pair(q, _):
        pass_body(q * 2, 0)
        pass_body(q * 2 + 1, 1)
        return 0

    lax.fori_loop(0, NPASS // 2, pair, 0)
    wb_copy(NPASS - 1, 1).wait()


@jax.jit
def _scatter_add(mem, val, idx):
    mesh = plsc.VectorSubcoreMesh(core_axis_name="c", subcore_axis_name="s")
    return pl.kernel(
        _body,
        out_type=jax.ShapeDtypeStruct((M, D), jnp.float32),
        mesh=mesh,
        compiler_params=pltpu.CompilerParams(needs_layout_passes=False,
                                             use_tc_tiling_on_sc=False),
        scratch_types=[
            pltpu.VMEM((SLICE,), jnp.int32),
            pltpu.VMEM((LISTCAP,), jnp.int32),
            pltpu.VMEM((LISTCAP,), jnp.int32),
            pltpu.VMEM((48,), jnp.int32),
        ] + [pltpu.VMEM((W,), jnp.int32)] * 8
          + [pltpu.VMEM((W, D), jnp.float32)] * 4
          + [
            pltpu.VMEM_SHARED((RANGE + 8, D), jnp.float32),
            pltpu.VMEM_SHARED((RANGE + 8, D), jnp.float32),
            pltpu.SemaphoreType.DMA,
            pltpu.SemaphoreType.DMA,
            pltpu.SemaphoreType.DMA,
            pltpu.SemaphoreType.DMA,
        ] + [pltpu.SemaphoreType.DMA] * 4,
    )(mem, val, idx)


def kernel(mem, val, idx):
    return _scatter_add(mem, val, idx)


# speedup vs baseline: 2.0146x; 2.0069x over previous
import jax
import jax.numpy as jnp
from jax import lax
from jax.experimental import pallas as pl
from jax.experimental.pallas import tpu as pltpu
from jax.experimental.pallas import tpu_sc as plsc

M = 524288
D = 64
B = 65536

NC = 2
NS = 16
NPASS = 32
RANGE = M // (NPASS * NC)
RSHIFT = 13
TROWS = RANGE // NS
SLICE = B // NS
CHUNKS = SLICE // 16
DUMPBIN = NPASS
LISTCAP = SLICE + 144
W = 128
NBUF = 4
WV = W // 16

SPROWS = 19000
ACCB = RANGE + 16
GARB = 2 * ACCB + 64
VPAD = 256


def _body(mem_hbm, val_hbm, idx_hbm, out_hbm, vso_hbm, idx_buf, lid_s,
          soff_all, fill, soff_chunk, vchunk,
          lw0, lw1, lw2, lw3, rb0, rb1, rb2, rb3,
          sp, ss_a, ss_b, ws_a, ws_b, gs0, gs1, gs2, gs3):
    c = lax.axis_index("c")
    s = lax.axis_index("s")

    ssems = (ss_a, ss_b)
    wsems = (ws_a, ws_b)
    lid_wins = (lw0, lw1, lw2, lw3)
    rows_bufs = (rb0, rb1, rb2, rb3)
    gsems = (gs0, gs1, gs2, gs3)

    pltpu.sync_copy(idx_hbm.at[pl.ds(s * SLICE, SLICE)], idx_buf)

    lanes = lax.iota(jnp.int32, 16)
    ones = jnp.ones((16,), jnp.int32)
    zeros = jnp.zeros((16,), jnp.int32)

    def bins_of(i):
        idxv = idx_buf[pl.ds(i * 16, 16)]
        r = idxv >> RSHIFT
        cm = (r & 1) ^ c
        pb = r >> 1
        return idxv, pb * (1 - cm) + DUMPBIN * cm

    fill[pl.ds(0, 16)] = zeros
    fill[pl.ds(16, 16)] = zeros
    fill[pl.ds(32, 16)] = zeros

    def count_chunk(i, _):
        _, binv = bins_of(i)
        plsc.addupdate_scatter(fill, [binv], ones)
        return 0

    with jax.named_scope("bincount"):
        lax.fori_loop(0, CHUNKS, count_chunk, 0)

    f0 = fill[pl.ds(0, 16)]
    e0 = plsc.cumsum(f0) - f0
    t0 = jnp.sum(f0)
    f1 = fill[pl.ds(16, 16)]
    e1 = t0 + plsc.cumsum(f1) - f1
    t1 = t0 + jnp.sum(f1)
    f2 = fill[pl.ds(32, 16)]
    e2 = t1 + plsc.cumsum(f2) - f2
    fill[pl.ds(0, 16)] = e0
    fill[pl.ds(16, 16)] = e1
    fill[pl.ds(32, 16)] = e2

    def scat_chunk(i, _):
        idxv, binv = bins_of(i)
        occ, _ = plsc.scan_count(binv)
        bf = plsc.load_gather(fill, [binv])
        off = bf + occ - 1
        plsc.store_scatter(lid_s, [off], idxv & (RANGE - 1))
        soff_all[pl.ds(i * 16, 16)] = off
        plsc.addupdate_scatter(fill, [binv], ones)
        return 0

    with jax.named_scope("binscat"):
        lax.fori_loop(0, CHUNKS, scat_chunk, 0)

    with jax.named_scope("reorder"):
        for wnd in range(4):
            @pl.when((s >> 2) == wnd)
            def _route():
                rbase = (s & 3) * SLICE

                def route_chunk(q, _):
                    pltpu.sync_copy(
                        val_hbm.at[pl.ds(s * SLICE + q * W, W)], vchunk)

                    def cp(j, _):
                        soff_chunk[pl.ds(j * 16, 16)] = rbase + \
                            soff_all[pl.ds(q * W + j * 16, 16)]
                        return 0

                    lax.fori_loop(0, WV, cp, 0)
                    pltpu.sync_copy(vchunk, sp.at[soff_chunk])
                    return 0

                lax.fori_loop(0, SLICE // W, route_chunk, 0)
                pltpu.sync_copy(
                    sp.at[pl.ds(rbase, SLICE)],
                    vso_hbm.at[pl.ds(c * B + s * SLICE, SLICE)])

            plsc.subcore_barrier()

    def stage_copy(p, half):
        base = (p * NC + c) * RANGE
        return pltpu.make_async_copy(
            mem_hbm.at[pl.ds(base + s * TROWS, TROWS)],
            sp.at[pl.ds(half * ACCB + s * TROWS, TROWS)], ssems[half])

    def wb_copy(p, half):
        base = (p * NC + c) * RANGE
        return pltpu.make_async_copy(
            sp.at[pl.ds(half * ACCB + s * TROWS, TROWS)],
            out_hbm.at[pl.ds(base + s * TROWS, TROWS)], wsems[half])

    stage_copy(0, 0).start()

    def endof(b):
        bv = zeros + b
        return plsc.load_gather(fill, [bv])[0]

    def adds(p, half):
        accbase = half * ACCB
        end = endof(p)
        start = endof(jnp.maximum(p - 1, 0)) * jnp.minimum(p, 1)
        head = start & 7
        wstart = start - head
        total = head + (end - start)
        nch = (total + (W - 1)) // W
        ngr = (nch + (NBUF - 1)) // NBUF

        def prep(k, b):
            def mrow(w, _):
                woff = pl.multiple_of(wstart + k * W + w * 16, 8)
                g = k * W + w * 16 + lanes
                valid = (1 - (((g - head) >> 31) & 1)) * \
                        (((g - total) >> 31) & 1)
                lw = lid_s[pl.ds(woff, 16)]
                lid_wins[b][pl.ds(w * 16, 16)] = \
                    valid * (accbase + lw) + (1 - valid) * GARB
                return 0

            lax.fori_loop(0, WV, mrow, 0)

        def grp(gidx, _):
            for b in range(NBUF):
                k = gidx * NBUF + b

                @pl.when(k < nch)
                def _fire(k=k, b=b):
                    prep(k, b)
                    woff = pl.multiple_of(wstart + k * W, 8)
                    pltpu.async_copy(
                        vso_hbm.at[pl.ds(c * B + s * SLICE + woff, W)],
                        rows_bufs[b], gsems[b])

            for b in range(NBUF):
                k = gidx * NBUF + b

                @pl.when(k < nch)
                def _drain(k=k, b=b):
                    woff = pl.multiple_of(wstart + k * W, 8)
                    pltpu.make_async_copy(
                        vso_hbm.at[pl.ds(c * B + s * SLICE + woff, W)],
                        rows_bufs[b], gsems[b]).wait()
                    pltpu.sync_copy(rows_bufs[b], sp.at[lid_wins[b]],
                                    add=True)
            return 0

        lax.fori_loop(0, ngr, grp, 0)

    def pass_body(p, half):
        other = 1 - half
        with jax.named_scope("wbwait"):
            @pl.when(p >= 1)
            def _():
                wb_copy(p - 1, other).wait()

            @pl.when(p + 1 < NPASS)
            def _():
                stage_copy(p + 1, other).start()
        with jax.named_scope("stwait"):
            stage_copy(p, half).wait()
        with jax.named_scope("bar1"):
            plsc.subcore_barrier()
        with jax.named_scope("adds"):
            adds(p, half)
        with jax.named_scope("bar2"):
            plsc.subcore_barrier()
        wb_copy(p, half).start()

    def pair(q, _):
        pass_body(q * 2, 0)
        pass_body(q * 2 + 1, 1)
        return 0

    lax.fori_loop(0, NPASS // 2, pair, 0)
    wb_copy(NPASS - 1, 1).wait()


@jax.jit
def _scatter_add(mem, val, idx):
    mesh = plsc.VectorSubcoreMesh(core_axis_name="c", subcore_axis_name="s")
    out, _ = pl.kernel(
        _body,
        out_type=[jax.ShapeDtypeStruct((M, D), jnp.float32),
                  jax.ShapeDtypeStruct((2 * B + VPAD, D), jnp.float32)],
        mesh=mesh,
        compiler_params=pltpu.CompilerParams(needs_layout_passes=False,
                                             use_tc_tiling_on_sc=False),
        scratch_types=[
            pltpu.VMEM((SLICE,), jnp.int32),
            pltpu.VMEM((LISTCAP,), jnp.int32),
            pltpu.VMEM((SLICE,), jnp.int32),
            pltpu.VMEM((48,), jnp.int32),
            pltpu.VMEM((W,), jnp.int32),
            pltpu.VMEM((W, D), jnp.float32),
        ] + [pltpu.VMEM((W,), jnp.int32)] * 4
          + [pltpu.VMEM((W, D), jnp.float32)] * 4
          + [
            pltpu.VMEM_SHARED((SPROWS, D), jnp.float32),
            pltpu.SemaphoreType.DMA,
            pltpu.SemaphoreType.DMA,
            pltpu.SemaphoreType.DMA,
            pltpu.SemaphoreType.DMA,
        ] + [pltpu.SemaphoreType.DMA] * 4,
    )(mem, val, idx)
    return out


def kernel(mem, val, idx):
    return _scatter_add(mem, val, idx)


# speedup vs baseline: 2.2168x; 1.1004x over previous
import jax
import jax.numpy as jnp
from jax import lax
from jax.experimental import pallas as pl
from jax.experimental.pallas import tpu as pltpu
from jax.experimental.pallas import tpu_sc as plsc

M = 524288
D = 64
B = 65536

NC = 2
NS = 16
NPASS = 32
RANGE = M // (NPASS * NC)
RSHIFT = 13
TROWS = RANGE // NS
SLICE = B // NS
CHUNKS = SLICE // 16
DUMPBIN = NPASS
LISTCAP = SLICE + 144
W = 128
NBUF = 2
WV = W // 16

SPROWS = 16500
ACCB = RANGE + 16
GARB = 2 * ACCB + 64
VPAD = 256
VQ = 256


def _body(mem_hbm, val_hbm, idx_hbm, out_hbm, vso_hbm, idx_buf, lid_s,
          soff_all, fill, soff_chunk, vchunk, vchunk2,
          lw0, lw1, rb0, rb1,
          sp, ss_a, ss_b, ws_a, ws_b, gs0, gs1, vs0, vs1):
    c = lax.axis_index("c")
    s = lax.axis_index("s")

    ssems = (ss_a, ss_b)
    wsems = (ws_a, ws_b)
    lid_wins = (lw0, lw1)
    rows_bufs = (rb0, rb1)
    gsems = (gs0, gs1)

    pltpu.sync_copy(idx_hbm.at[pl.ds(s * SLICE, SLICE)], idx_buf)

    lanes = lax.iota(jnp.int32, 16)
    ones = jnp.ones((16,), jnp.int32)
    zeros = jnp.zeros((16,), jnp.int32)

    def bins_of(i):
        idxv = idx_buf[pl.ds(i * 16, 16)]
        r = idxv >> RSHIFT
        cm = (r & 1) ^ c
        pb = r >> 1
        return idxv, pb * (1 - cm) + DUMPBIN * cm

    fill[pl.ds(0, 16)] = zeros
    fill[pl.ds(16, 16)] = zeros
    fill[pl.ds(32, 16)] = zeros

    def count_chunk(i, _):
        _, binv = bins_of(i)
        plsc.addupdate_scatter(fill, [binv], ones)
        return 0

    with jax.named_scope("bincount"):
        lax.fori_loop(0, CHUNKS, count_chunk, 0)

    f0 = fill[pl.ds(0, 16)]
    e0 = plsc.cumsum(f0) - f0
    t0 = jnp.sum(f0)
    f1 = fill[pl.ds(16, 16)]
    e1 = t0 + plsc.cumsum(f1) - f1
    t1 = t0 + jnp.sum(f1)
    f2 = fill[pl.ds(32, 16)]
    e2 = t1 + plsc.cumsum(f2) - f2
    fill[pl.ds(0, 16)] = e0
    fill[pl.ds(16, 16)] = e1
    fill[pl.ds(32, 16)] = e2

    def scat_chunk(i, _):
        idxv, binv = bins_of(i)
        occ, _ = plsc.scan_count(binv)
        bf = plsc.load_gather(fill, [binv])
        off = bf + occ - 1
        plsc.store_scatter(lid_s, [off], idxv & (RANGE - 1))
        soff_all[pl.ds(i * 16, 16)] = off
        plsc.addupdate_scatter(fill, [binv], ones)
        return 0

    with jax.named_scope("binscat"):
        lax.fori_loop(0, CHUNKS, scat_chunk, 0)

    with jax.named_scope("reorder"):
        NQ = SLICE // VQ
        for wnd in range(4):
            @pl.when((s >> 2) == wnd)
            def _route():
                rbase = (s & 3) * SLICE
                vbufs = (vchunk, vchunk2)
                vsems = (vs0, vs1)

                def vfetch(q, h):
                    return pltpu.make_async_copy(
                        val_hbm.at[pl.ds(s * SLICE + q * VQ, VQ)],
                        vbufs[h], vsems[h])

                vfetch(0, 0).start()

                def route_q(q, h):
                    @pl.when(q + 1 < NQ)
                    def _():
                        vfetch(q + 1, 1 - h).start()
                    vfetch(q, h).wait()
                    for sub in range(VQ // W):
                        def cp(j, _, sub=sub):
                            soff_chunk[pl.ds(j * 16, 16)] = rbase + \
                                soff_all[pl.ds(q * VQ + sub * W + j * 16,
                                               16)]
                            return 0

                        lax.fori_loop(0, WV, cp, 0)
                        pltpu.sync_copy(
                            vbufs[h].at[pl.ds(sub * W, W)],
                            sp.at[soff_chunk])

                def route_pair(qq, _):
                    route_q(qq * 2, 0)
                    route_q(qq * 2 + 1, 1)
                    return 0

                lax.fori_loop(0, NQ // 2, route_pair, 0)
                pltpu.sync_copy(
                    sp.at[pl.ds(rbase, SLICE)],
                    vso_hbm.at[pl.ds(c * B + s * SLICE, SLICE)])

            plsc.subcore_barrier()

    def stage_copy(p, half):
        base = (p * NC + c) * RANGE
        return pltpu.make_async_copy(
            mem_hbm.at[pl.ds(base + s * TROWS, TROWS)],
            sp.at[pl.ds(half * ACCB + s * TROWS, TROWS)], ssems[half])

    def wb_copy(p, half):
        base = (p * NC + c) * RANGE
        return pltpu.make_async_copy(
            sp.at[pl.ds(half * ACCB + s * TROWS, TROWS)],
            out_hbm.at[pl.ds(base + s * TROWS, TROWS)], wsems[half])

    stage_copy(0, 0).start()

    def endof(b):
        bv = zeros + b
        return plsc.load_gather(fill, [bv])[0]

    def adds(p, half):
        accbase = half * ACCB
        end = endof(p)
        start = endof(jnp.maximum(p - 1, 0)) * jnp.minimum(p, 1)
        head = start & 7
        wstart = start - head
        total = head + (end - start)
        nch = (total + (W - 1)) // W
        ngr = (nch + (NBUF - 1)) // NBUF

        def prep(k, b):
            def mrow(w, _):
                woff = pl.multiple_of(wstart + k * W + w * 16, 8)
                g = k * W + w * 16 + lanes
                valid = (1 - (((g - head) >> 31) & 1)) * \
                        (((g - total) >> 31) & 1)
                lw = lid_s[pl.ds(woff, 16)]
                lid_wins[b][pl.ds(w * 16, 16)] = \
                    valid * (accbase + lw) + (1 - valid) * GARB
                return 0

            lax.fori_loop(0, WV, mrow, 0)

        def grp(gidx, _):
            for b in range(NBUF):
                k = gidx * NBUF + b

                @pl.when(k < nch)
                def _fire(k=k, b=b):
                    prep(k, b)
                    woff = pl.multiple_of(wstart + k * W, 8)
                    pltpu.async_copy(
                        vso_hbm.at[pl.ds(c * B + s * SLICE + woff, W)],
                        rows_bufs[b], gsems[b])

            for b in range(NBUF):
                k = gidx * NBUF + b

                @pl.when(k < nch)
                def _drain(k=k, b=b):
                    woff = pl.multiple_of(wstart + k * W, 8)
                    pltpu.make_async_copy(
                        vso_hbm.at[pl.ds(c * B + s * SLICE + woff, W)],
                        rows_bufs[b], gsems[b]).wait()
                    pltpu.sync_copy(rows_bufs[b], sp.at[lid_wins[b]],
                                    add=True)
            return 0

        lax.fori_loop(0, ngr, grp, 0)

    def pass_body(p, half):
        other = 1 - half
        with jax.named_scope("wbwait"):
            @pl.when(p >= 1)
            def _():
                wb_copy(p - 1, other).wait()

            @pl.when(p + 1 < NPASS)
            def _():
                stage_copy(p + 1, other).start()
        with jax.named_scope("stwait"):
            stage_copy(p, half).wait()
        with jax.named_scope("bar1"):
            plsc.subcore_barrier()
        with jax.named_scope("adds"):
            adds(p, half)
        with jax.named_scope("bar2"):
            plsc.subcore_barrier()
        wb_copy(p, half).start()

    def pair(q, _):
        pass_body(q * 2, 0)
        pass_body(q * 2 + 1, 1)
        return 0

    lax.fori_loop(0, NPASS // 2, pair, 0)
    wb_copy(NPASS - 1, 1).wait()


@jax.jit
def _scatter_add(mem, val, idx):
    mesh = plsc.VectorSubcoreMesh(core_axis_name="c", subcore_axis_name="s")
    out, _ = pl.kernel(
        _body,
        out_type=[jax.ShapeDtypeStruct((M, D), jnp.float32),
                  jax.ShapeDtypeStruct((2 * B + VPAD, D), jnp.float32)],
        mesh=mesh,
        compiler_params=pltpu.CompilerParams(needs_layout_passes=False,
                                             use_tc_tiling_on_sc=False),
        scratch_types=[
            pltpu.VMEM((SLICE,), jnp.int32),
            pltpu.VMEM((LISTCAP,), jnp.int32),
            pltpu.VMEM((SLICE,), jnp.int32),
            pltpu.VMEM((48,), jnp.int32),
            pltpu.VMEM((W,), jnp.int32),
            pltpu.VMEM((VQ, D), jnp.float32),
            pltpu.VMEM((VQ, D), jnp.float32),
        ] + [pltpu.VMEM((W,), jnp.int32)] * 2
          + [pltpu.VMEM((W, D), jnp.float32)] * 2
          + [
            pltpu.VMEM_SHARED((SPROWS, D), jnp.float32),
            pltpu.SemaphoreType.DMA,
            pltpu.SemaphoreType.DMA,
            pltpu.SemaphoreType.DMA,
            pltpu.SemaphoreType.DMA,
        ] + [pltpu.SemaphoreType.DMA] * 2
          + [pltpu.SemaphoreType.DMA] * 2,
    )(mem, val, idx)
    return out


def kernel(mem, val, idx):
    return _scatter_add(mem, val, idx)


# speedup vs baseline: 2.2189x; 1.0009x over previous
import functools

import jax
import jax.numpy as jnp
from jax import lax
from jax.experimental import layout as jlayout
from jax.experimental import pallas as pl
from jax.experimental.pallas import tpu as pltpu
from jax.experimental.pallas import tpu_sc as plsc

M = 524288
D = 64
B = 65536

NC = 2
NS = 16
NPASS = 32
RANGE = M // (NPASS * NC)
RSHIFT = 13
TROWS = RANGE // NS
SLICE = B // NS
CHUNKS = SLICE // 16
DUMPBIN = NPASS
LISTCAP = SLICE + 144
W = 128
NBUF = 2
WV = W // 16

SPROWS = 16420
ACCB = RANGE + 16
GARB = 2 * ACCB + 64
VPAD = 256
VQ = 256


def _body(mem_hbm, val_hbm, idx_hbm, out_hbm, vso_hbm, idx_buf, lid_s,
          soff_all, fill, soff_chunk, vchunk, vchunk2,
          lw0, lw1, rb0, rb1,
          sp, ss_a, ss_b, ws_a, ws_b, gs0, gs1, vs0, vs1):
    c = lax.axis_index("c")
    s = lax.axis_index("s")

    ssems = (ss_a, ss_b)
    wsems = (ws_a, ws_b)
    lid_wins = (lw0, lw1)
    rows_bufs = (rb0, rb1)
    gsems = (gs0, gs1)

    pltpu.sync_copy(idx_hbm.at[pl.ds(s * SLICE, SLICE)], idx_buf)

    lanes = lax.iota(jnp.int32, 16)
    ones = jnp.ones((16,), jnp.int32)
    zeros = jnp.zeros((16,), jnp.int32)

    def bins_of(i):
        idxv = idx_buf[pl.ds(i * 16, 16)]
        r = idxv >> RSHIFT
        cm = (r & 1) ^ c
        pb = r >> 1
        return idxv, pb * (1 - cm) + DUMPBIN * cm

    fill[pl.ds(0, 16)] = zeros
    fill[pl.ds(16, 16)] = zeros
    fill[pl.ds(32, 16)] = zeros

    def count_chunk(i, _):
        _, binv = bins_of(i)
        plsc.addupdate_scatter(fill, [binv], ones)
        return 0

    with jax.named_scope("bincount"):
        lax.fori_loop(0, CHUNKS, count_chunk, 0)

    f0 = fill[pl.ds(0, 16)]
    e0 = plsc.cumsum(f0) - f0
    t0 = jnp.sum(f0)
    f1 = fill[pl.ds(16, 16)]
    e1 = t0 + plsc.cumsum(f1) - f1
    t1 = t0 + jnp.sum(f1)
    f2 = fill[pl.ds(32, 16)]
    e2 = t1 + plsc.cumsum(f2) - f2
    fill[pl.ds(0, 16)] = e0
    fill[pl.ds(16, 16)] = e1
    fill[pl.ds(32, 16)] = e2

    def scat_chunk(i, _):
        idxv, binv = bins_of(i)
        occ, _ = plsc.scan_count(binv)
        bf = plsc.load_gather(fill, [binv])
        off = bf + occ - 1
        plsc.store_scatter(lid_s, [off], idxv & (RANGE - 1))
        soff_all[pl.ds(i * 16, 16)] = off
        plsc.addupdate_scatter(fill, [binv], ones)
        return 0

    with jax.named_scope("binscat"):
        lax.fori_loop(0, CHUNKS, scat_chunk, 0)

    with jax.named_scope("reorder"):
        NQ = SLICE // VQ
        for wnd in range(4):
            @pl.when((s >> 2) == wnd)
            def _route():
                rbase = (s & 3) * SLICE
                vbufs = (vchunk, vchunk2)
                vsems = (vs0, vs1)

                def vfetch(q, h):
                    return pltpu.make_async_copy(
                        val_hbm.at[pl.ds(s * SLICE + q * VQ, VQ)],
                        vbufs[h], vsems[h])

                def route_q(q, h):
                    @pl.when(q + 1 < NQ)
                    def _():
                        vfetch(q + 1, 1 - h).start()
                    vfetch(q, h).wait()
                    for sub in range(VQ // W):
                        def cp(j, _, sub=sub):
                            soff_chunk[pl.ds(j * 16, 16)] = rbase + \
                                soff_all[pl.ds(q * VQ + sub * W + j * 16,
                                               16)]
                            return 0

                        lax.fori_loop(0, WV, cp, 0)
                        pltpu.sync_copy(
                            vbufs[h].at[pl.ds(sub * W, W)],
                            sp.at[soff_chunk])

                vfetch(0, 0).start()

                def route_pair(qq, _):
                    route_q(qq * 2, 0)
                    route_q(qq * 2 + 1, 1)
                    return 0

                lax.fori_loop(0, NQ // 2, route_pair, 0)
                pltpu.sync_copy(
                    sp.at[pl.ds(rbase, SLICE)],
                    vso_hbm.at[pl.ds(c * B + s * SLICE, SLICE)])

            plsc.subcore_barrier()

    def stage_copy(p, half):
        base = (p * NC + c) * RANGE
        return pltpu.make_async_copy(
            mem_hbm.at[pl.ds(base + s * TROWS, TROWS)],
            sp.at[pl.ds(half * ACCB + s * TROWS, TROWS)], ssems[half])

    def wb_copy(p, half):
        base = (p * NC + c) * RANGE
        return pltpu.make_async_copy(
            sp.at[pl.ds(half * ACCB + s * TROWS, TROWS)],
            out_hbm.at[pl.ds(base + s * TROWS, TROWS)], wsems[half])

    stage_copy(0, 0).start()

    def endof(b):
        bv = zeros + b
        return plsc.load_gather(fill, [bv])[0]

    def adds(p, half):
        accbase = half * ACCB
        end = endof(p)
        start = endof(jnp.maximum(p - 1, 0)) * jnp.minimum(p, 1)
        head = start & 7
        wstart = start - head
        total = head + (end - start)
        nch = (total + (W - 1)) // W
        ngr = (nch + (NBUF - 1)) // NBUF

        def prep(k, b):
            def mrow(w, _):
                woff = pl.multiple_of(wstart + k * W + w * 16, 8)
                g = k * W + w * 16 + lanes
                valid = (1 - (((g - head) >> 31) & 1)) * \
                        (((g - total) >> 31) & 1)
                lw = lid_s[pl.ds(woff, 16)]
                lid_wins[b][pl.ds(w * 16, 16)] = \
                    valid * (accbase + lw) + (1 - valid) * GARB
                return 0

            lax.fori_loop(0, WV, mrow, 0)

        def grp(gidx, _):
            for b in range(NBUF):
                k = gidx * NBUF + b

                @pl.when(k < nch)
                def _fire(k=k, b=b):
                    prep(k, b)
                    woff = pl.multiple_of(wstart + k * W, 8)
                    pltpu.async_copy(
                        vso_hbm.at[pl.ds(c * B + s * SLICE + woff, W)],
                        rows_bufs[b], gsems[b])

            for b in range(NBUF):
                k = gidx * NBUF + b

                @pl.when(k < nch)
                def _drain(k=k, b=b):
                    woff = pl.multiple_of(wstart + k * W, 8)
                    pltpu.make_async_copy(
                        vso_hbm.at[pl.ds(c * B + s * SLICE + woff, W)],
                        rows_bufs[b], gsems[b]).wait()
                    pltpu.sync_copy(rows_bufs[b], sp.at[lid_wins[b]],
                                    add=True)
            return 0

        lax.fori_loop(0, ngr, grp, 0)

    def pass_body(p, half):
        other = 1 - half
        with jax.named_scope("wbwait"):
            @pl.when(p >= 1)
            def _():
                wb_copy(p - 1, other).wait()

            @pl.when(p + 1 < NPASS)
            def _():
                stage_copy(p + 1, other).start()
        with jax.named_scope("stwait"):
            stage_copy(p, half).wait()
        with jax.named_scope("bar1"):
            plsc.subcore_barrier()
        with jax.named_scope("adds"):
            adds(p, half)
        with jax.named_scope("bar2"):
            plsc.subcore_barrier()
        wb_copy(p, half).start()

    def pair(q, _):
        pass_body(q * 2, 0)
        pass_body(q * 2 + 1, 1)
        return 0

    lax.fori_loop(0, NPASS // 2, pair, 0)
    wb_copy(NPASS - 1, 1).wait()


@jax.jit
def _scatter_add(mem, val, idx):
    mesh = plsc.VectorSubcoreMesh(core_axis_name="c", subcore_axis_name="s")
    out, _ = pl.kernel(
        _body,
        out_type=[jax.ShapeDtypeStruct((M, D), jnp.float32),
                  jax.ShapeDtypeStruct((2 * B + VPAD, D), jnp.float32)],
        mesh=mesh,
        compiler_params=pltpu.CompilerParams(needs_layout_passes=False,
                                             use_tc_tiling_on_sc=False),
        scratch_types=[
            pltpu.VMEM((SLICE,), jnp.int32),
            pltpu.VMEM((LISTCAP,), jnp.int32),
            pltpu.VMEM((SLICE,), jnp.int32),
            pltpu.VMEM((48,), jnp.int32),
            pltpu.VMEM((W,), jnp.int32),
            pltpu.VMEM((VQ, D), jnp.float32),
            pltpu.VMEM((VQ, D), jnp.float32),
        ] + [pltpu.VMEM((W,), jnp.int32)] * 2
          + [pltpu.VMEM((W, D), jnp.float32)] * 2
          + [
            pltpu.VMEM_SHARED((SPROWS, D), jnp.float32),
            pltpu.SemaphoreType.DMA,
            pltpu.SemaphoreType.DMA,
            pltpu.SemaphoreType.DMA,
            pltpu.SemaphoreType.DMA,
        ] + [pltpu.SemaphoreType.DMA] * 2
          + [pltpu.SemaphoreType.DMA] * 2,
    )(mem, val, idx)
    return jlayout.with_layout_constraint(
        out, jlayout.Layout(major_to_minor=(1, 0)))


def kernel(mem, val, idx):
    return _scatter_add(mem, val, idx)
